# baseline (speedup 1.0000x reference)
import functools
import math

import jax
import jax.numpy as jnp
from jax import lax
from jax.experimental import pallas as pl
from jax.experimental.pallas import tpu as pltpu
from jax.experimental.pallas import tpu_sc as plsc

N = 10000
E = 320000
D = 128
DE = 16
H = 16
DOUT = 128
COS = math.cos(math.pi / 8)
SIN = math.sin(math.pi / 8)
INV_SQRT_NN = 1.0 / math.sqrt(32.0)
INV_SQRT_H = 1.0 / math.sqrt(float(H))

NC, NS = 2, 16
NW = NC * NS
EPW = E // NW
CHUNK = 40
NCHUNK = EPW // CHUNK
ZROWS = 40
NZB = N // ZROWS


def _self_body(x_ref, w_ref, nf_ref, nso_ref):
    t = jnp.dot(x_ref[...], w_ref[...], preferred_element_type=jnp.float32)
    nf_ref[...] = t[:, :D]
    nso_ref[...] = t[:, D:]


def _self_interaction(node_input, W_self):
    B = 1000
    return pl.pallas_call(
        _self_body,
        grid=(N // B,),
        in_specs=[
            pl.BlockSpec((B, D), lambda i: (i, 0)),
            pl.BlockSpec((D, D + DOUT), lambda i: (0, 0)),
        ],
        out_specs=[
            pl.BlockSpec((B, D), lambda i: (i, 0)),
            pl.BlockSpec((B, DOUT), lambda i: (i, 0)),
        ],
        out_shape=[
            jax.ShapeDtypeStruct((N, D), jnp.float32),
            jax.ShapeDtypeStruct((N, DOUT), jnp.float32),
        ],
    )(node_input, W_self)


def _mix_body(esa_ref, ea_ref, w1_ref, w2_ref, rep_ref, til_ref, wtp_ref,
              mix_ref):
    rows = esa_ref.shape[0]
    w = jax.nn.gelu(jnp.dot(esa_ref[...], w1_ref[...],
                            preferred_element_type=jnp.float32))
    w = jax.nn.gelu(jnp.dot(w, w2_ref[...],
                            preferred_element_type=jnp.float32))
    ea = ea_ref[...]
    for i in range(8):
        w_i = w[:, 16 * i:16 * i + 16].astype(jnp.bfloat16)
        ea_i = ea[:, 16 * i:16 * i + 16].astype(jnp.bfloat16)
        a_i = (jnp.dot(w_i, rep_ref[...], preferred_element_type=jnp.float32)
               * jnp.dot(ea_i, til_ref[...],
                         preferred_element_type=jnp.float32)
               ).astype(jnp.bfloat16)
        mix_ref[rows * i:rows * (i + 1), :] = jnp.dot(
            a_i, wtp_ref[...], preferred_element_type=jnp.float32)


def _edge_mix(esa_p, ea_p, W1b, W2b, rep, til, W2s):
    B = 6400
    return pl.pallas_call(
        _mix_body,
        grid=(E // B,),
        in_specs=[
            pl.BlockSpec((B // 8, 128), lambda i: (i, 0)),
            pl.BlockSpec((B // 8, 128), lambda i: (i, 0)),
            pl.BlockSpec((128, 128), lambda i: (0, 0)),
            pl.BlockSpec((128, 128), lambda i: (0, 0)),
            pl.BlockSpec((H, H * DE), lambda i: (0, 0)),
            pl.BlockSpec((DE, H * DE), lambda i: (0, 0)),
            pl.BlockSpec((H * DE, D), lambda i: (0, 0)),
        ],
        out_specs=pl.BlockSpec((B, D), lambda i: (i, 0)),
        out_shape=jax.ShapeDtypeStruct((E, D), jnp.float32),
    )(esa_p, ea_p, W1b, W2b, rep, til, W2s)


def _sc_body(nf_hbm, mix_hbm, src_hbm, dst_hbm, out_hbm,
             src_v, dst_v, rows_v, mix_v, src_v2, dst_v2, rows_v2, mix_v2,
             agg_sh, sem_g, sem_m, sem_s, sem_d, sem_g2, sem_m2, sem_s2,
             sem_d2):
    c = lax.axis_index("c")
    s = lax.axis_index("s")
    wid = c * NS + s

    def zrow(r, carry):
        for g in range(D // 16):
            rows_v[r, pl.ds(16 * g, 16)] = jnp.zeros((16,), jnp.float32)
        return carry

    lax.fori_loop(0, ZROWS, zrow, 0)
    for k in range((NZB + NS - 1) // NS):
        b = s + NS * k
        @pl.when(b < NZB)
        def _():
            pltpu.sync_copy(rows_v, agg_sh.at[pl.ds(b * ZROWS, ZROWS)])
    plsc.subcore_barrier()

    src_b = [src_v, src_v2]
    dst_b = [dst_v, dst_v2]
    rows_b = [rows_v, rows_v2]
    mix_b = [mix_v, mix_v2]
    sg = [sem_g, sem_g2]
    sm = [sem_m, sem_m2]
    ss = [sem_s, sem_s2]
    sd = [sem_d, sem_d2]

    def issue_pre(j, p):
        pltpu.async_copy(src_hbm.at[wid, j], src_b[p], ss[p])
        pltpu.async_copy(dst_hbm.at[wid, j], dst_b[p], sd[p])
        pltpu.async_copy(mix_hbm.at[pl.ds(wid * EPW + j * CHUNK, CHUNK)],
                         mix_b[p], sm[p])

    def wait_src(p):
        pltpu.make_async_copy(src_hbm.at[wid, 0], src_b[p], ss[p]).wait()

    def issue_gather(j, p):
        pltpu.async_copy(nf_hbm.at[src_b[p].at[0]], rows_b[p], sg[p])

    issue_pre(0, 0)
    issue_pre(1, 1)
    wait_src(0)
    issue_gather(0, 0)

    def sub(j, p, q):
        @pl.when(j + 1 < NCHUNK)
        def _():
            wait_src(q)
            issue_gather(j + 1, q)
        pltpu.make_async_copy(nf_hbm.at[src_b[p].at[0]], rows_b[p],
                              sg[p]).wait()
        pltpu.make_async_copy(mix_hbm.at[pl.ds(0, CHUNK)], mix_b[p],
                              sm[p]).wait()
        for r in range(CHUNK):
            for g in range(D // 16):
                sl = pl.ds(16 * g, 16)
                rows_b[p][r, sl] = rows_b[p][r, sl] * mix_b[p][r, sl]
        pltpu.make_async_copy(dst_hbm.at[wid, 0], dst_b[p], sd[p]).wait()
        pltpu.sync_copy(rows_b[p], agg_sh.at[dst_b[p].at[0]], add=True)
        @pl.when(j + 2 < NCHUNK)
        def _():
            issue_pre(j + 2, p)

    def pair(t, carry):
        sub(2 * t, 0, 1)
        sub(2 * t + 1, 1, 0)
        return carry

    lax.fori_loop(0, NCHUNK // 2, pair, 0)
    plsc.subcore_barrier()

    for k in range((NZB + NS - 1) // NS):
        b = s + NS * k
        @pl.when(b < NZB)
        def _():
            pltpu.sync_copy(agg_sh.at[pl.ds(b * ZROWS, ZROWS)], rows_v)
            pltpu.sync_copy(rows_v, out_hbm.at[c, pl.ds(b * ZROWS, ZROWS)])


_sc_scatter = functools.partial(
    pl.kernel,
    out_type=jax.ShapeDtypeStruct((NC, N, D), jnp.float32),
    mesh=plsc.VectorSubcoreMesh(core_axis_name="c", subcore_axis_name="s"),
    scratch_types=(
        [pltpu.VMEM((1, CHUNK), jnp.int32),
         pltpu.VMEM((1, CHUNK), jnp.int32),
         pltpu.VMEM((CHUNK, D), jnp.float32),
         pltpu.VMEM((CHUNK, D), jnp.float32)] * 2
        + [pltpu.VMEM_SHARED((N, D), jnp.float32)]
        + [pltpu.SemaphoreType.DMA] * 8
    ),
)(_sc_body)


def _post_body(nso_ref, a0_ref, a1_ref, w_ref, o_ref):
    agg = a0_ref[...] + a1_ref[...]
    o_ref[...] = (COS * nso_ref[...]
                  + jnp.dot(agg, w_ref[...],
                            preferred_element_type=jnp.float32))


def _post(nso, agg0, agg1, W_out_scaled):
    B = 1000
    return pl.pallas_call(
        _post_body,
        grid=(N // B,),
        in_specs=[
            pl.BlockSpec((B, DOUT), lambda i: (i, 0)),
            pl.BlockSpec((B, D), lambda i: (i, 0)),
            pl.BlockSpec((B, D), lambda i: (i, 0)),
            pl.BlockSpec((D, DOUT), lambda i: (0, 0)),
        ],
        out_specs=pl.BlockSpec((B, DOUT), lambda i: (i, 0)),
        out_shape=jax.ShapeDtypeStruct((N, DOUT), jnp.float32),
    )(nso, agg0, agg1, W_out_scaled)


def kernel(node_input, edge_src, edge_dst, edge_attr, edge_scalar_attr,
           W_self, W_mlp1, W_mlp2, W_tp, W_out):
    nf, nso = _self_interaction(node_input, W_self)

    eye = jnp.eye(H, dtype=jnp.bfloat16)
    rep = jnp.repeat(eye, DE, axis=1)
    til = jnp.tile(jnp.eye(DE, dtype=jnp.bfloat16), (1, H))
    W2s = (jnp.transpose(W_tp, (0, 2, 1)).reshape(H * DE, D)
           * INV_SQRT_H).astype(jnp.bfloat16)
    W1b = jnp.kron(jnp.eye(8, dtype=jnp.float32), W_mlp1)
    W2b = jnp.kron(jnp.eye(8, dtype=jnp.float32), W_mlp2)

    mix = _edge_mix(edge_scalar_attr.reshape(E // 8, 128),
                    edge_attr.reshape(E // 8, 128),
                    W1b, W2b, rep, til, W2s)

    MB = 6400
    perm = jnp.arange(E, dtype=jnp.int32).reshape(
        E // MB, MB // 8, 8).transpose(0, 2, 1).reshape(E)
    src4d = edge_src[perm].reshape(NW, NCHUNK, 1, CHUNK)
    dst4d = edge_dst[perm].reshape(NW, NCHUNK, 1, CHUNK)
    aggs = _sc_scatter(nf, mix, src4d, dst4d)

    return _post(nso, aggs[0], aggs[1], W_out * (SIN * INV_SQRT_NN))


# speedup vs baseline: 1.3297x; 1.3297x over previous
import functools
import math

import jax
import jax.numpy as jnp
from jax import lax
from jax.experimental import pallas as pl
from jax.experimental.pallas import tpu as pltpu
from jax.experimental.pallas import tpu_sc as plsc

N = 10000
E = 320000
D = 128
DE = 16
H = 16
DOUT = 128
COS = math.cos(math.pi / 8)
SIN = math.sin(math.pi / 8)
INV_SQRT_NN = 1.0 / math.sqrt(32.0)
INV_SQRT_H = 1.0 / math.sqrt(float(H))

NC, NS = 2, 16
NW = NC * NS
EPW = E // NW
CHUNK = 40
NCHUNK = EPW // CHUNK
ZROWS = 40
NZB = N // ZROWS


def _self_body(x_ref, w_ref, nf_ref, nso_ref):
    t = jnp.dot(x_ref[...], w_ref[...], preferred_element_type=jnp.float32)
    nf_ref[...] = t[:, :D]
    nso_ref[...] = t[:, D:]


def _self_interaction(node_input, W_self):
    B = 1000
    return pl.pallas_call(
        _self_body,
        grid=(N // B,),
        in_specs=[
            pl.BlockSpec((B, D), lambda i: (i, 0)),
            pl.BlockSpec((D, D + DOUT), lambda i: (0, 0)),
        ],
        out_specs=[
            pl.BlockSpec((B, D), lambda i: (i, 0)),
            pl.BlockSpec((B, DOUT), lambda i: (i, 0)),
        ],
        out_shape=[
            jax.ShapeDtypeStruct((N, D), jnp.float32),
            jax.ShapeDtypeStruct((N, DOUT), jnp.float32),
        ],
    )(node_input, W_self)


def _mix_body(esa_ref, ea_ref, w1_ref, w2_ref, rep_ref, til_ref, wtp_ref,
              mix_ref):
    rows = esa_ref.shape[0]
    w = jax.nn.gelu(jnp.dot(esa_ref[...], w1_ref[...],
                            preferred_element_type=jnp.float32))
    w = jax.nn.gelu(jnp.dot(w, w2_ref[...],
                            preferred_element_type=jnp.float32))
    ea = ea_ref[...]
    for i in range(8):
        w_i = w[:, 16 * i:16 * i + 16].astype(jnp.bfloat16)
        ea_i = ea[:, 16 * i:16 * i + 16].astype(jnp.bfloat16)
        a_i = (jnp.dot(w_i, rep_ref[...], preferred_element_type=jnp.float32)
               * jnp.dot(ea_i, til_ref[...],
                         preferred_element_type=jnp.float32)
               ).astype(jnp.bfloat16)
        mix_ref[rows * i:rows * (i + 1), :] = jnp.dot(
            a_i, wtp_ref[...], preferred_element_type=jnp.float32)


def _edge_mix(esa_p, ea_p, W1b, W2b, rep, til, W2s):
    B = 6400
    return pl.pallas_call(
        _mix_body,
        grid=(E // B,),
        in_specs=[
            pl.BlockSpec((B // 8, 128), lambda i: (i, 0)),
            pl.BlockSpec((B // 8, 128), lambda i: (i, 0)),
            pl.BlockSpec((128, 128), lambda i: (0, 0)),
            pl.BlockSpec((128, 128), lambda i: (0, 0)),
            pl.BlockSpec((H, H * DE), lambda i: (0, 0)),
            pl.BlockSpec((DE, H * DE), lambda i: (0, 0)),
            pl.BlockSpec((H * DE, D), lambda i: (0, 0)),
        ],
        out_specs=pl.BlockSpec((B, D), lambda i: (i, 0)),
        out_shape=jax.ShapeDtypeStruct((E, D), jnp.float32),
    )(esa_p, ea_p, W1b, W2b, rep, til, W2s)


def _sc_body(nf_hbm, mix_hbm, src_hbm, dst_hbm, out_hbm,
             src_v, dst_v, rows_v, mix_v, src_v2, dst_v2, rows_v2, mix_v2,
             agg_sh, sem_g, sem_m, sem_s, sem_d, sem_g2, sem_m2, sem_s2,
             sem_d2):
    c = lax.axis_index("c")
    s = lax.axis_index("s")
    wid = c * NS + s

    def zrow(r, carry):
        for g in range(D // 16):
            rows_v[r, pl.ds(16 * g, 16)] = jnp.zeros((16,), jnp.float32)
        return carry

    lax.fori_loop(0, ZROWS, zrow, 0)
    for k in range((NZB + NS - 1) // NS):
        b = s + NS * k
        @pl.when(b < NZB)
        def _():
            pltpu.sync_copy(rows_v, agg_sh.at[pl.ds(b * ZROWS, ZROWS)])
    plsc.subcore_barrier()

    src_b = [src_v, src_v2]
    dst_b = [dst_v, dst_v2]
    rows_b = [rows_v, rows_v2]
    mix_b = [mix_v, mix_v2]
    sg = [sem_g, sem_g2]
    sm = [sem_m, sem_m2]
    ss = [sem_s, sem_s2]
    sd = [sem_d, sem_d2]

    def issue_pre(j, p):
        pltpu.async_copy(src_hbm.at[wid, j], src_b[p], ss[p])
        pltpu.async_copy(dst_hbm.at[wid, j], dst_b[p], sd[p])
        pltpu.async_copy(mix_hbm.at[pl.ds(wid * EPW + j * CHUNK, CHUNK)],
                         mix_b[p], sm[p])

    def wait_src(p):
        pltpu.make_async_copy(src_hbm.at[wid, 0], src_b[p], ss[p]).wait()

    def issue_gather(j, p):
        pltpu.async_copy(nf_hbm.at[src_b[p].at[0]], rows_b[p], sg[p])

    issue_pre(0, 0)
    issue_pre(1, 1)
    wait_src(0)
    issue_gather(0, 0)

    def sub(j, p, q):
        @pl.when(j + 1 < NCHUNK)
        def _():
            wait_src(q)
            issue_gather(j + 1, q)
        pltpu.make_async_copy(nf_hbm.at[src_b[p].at[0]], rows_b[p],
                              sg[p]).wait()
        pltpu.make_async_copy(mix_hbm.at[pl.ds(0, CHUNK)], mix_b[p],
                              sm[p]).wait()
        def mrow(r, inner):
            for g in range(D // 16):
                sl = pl.ds(16 * g, 16)
                rows_b[p][r, sl] = rows_b[p][r, sl] * mix_b[p][r, sl]
            return inner

        lax.fori_loop(0, CHUNK, mrow, 0)
        pltpu.make_async_copy(dst_hbm.at[wid, 0], dst_b[p], sd[p]).wait()
        pltpu.sync_copy(rows_b[p], agg_sh.at[dst_b[p].at[0]], add=True)
        @pl.when(j + 2 < NCHUNK)
        def _():
            issue_pre(j + 2, p)

    def pair(t, carry):
        sub(2 * t, 0, 1)
        sub(2 * t + 1, 1, 0)
        return carry

    lax.fori_loop(0, NCHUNK // 2, pair, 0)
    plsc.subcore_barrier()

    for k in range((NZB + NS - 1) // NS):
        b = s + NS * k
        @pl.when(b < NZB)
        def _():
            pltpu.sync_copy(agg_sh.at[pl.ds(b * ZROWS, ZROWS)], rows_v)
            pltpu.sync_copy(rows_v, out_hbm.at[c, pl.ds(b * ZROWS, ZROWS)])


_sc_scatter = functools.partial(
    pl.kernel,
    out_type=jax.ShapeDtypeStruct((NC, N, D), jnp.float32),
    mesh=plsc.VectorSubcoreMesh(core_axis_name="c", subcore_axis_name="s"),
    scratch_types=(
        [pltpu.VMEM((1, CHUNK), jnp.int32),
         pltpu.VMEM((1, CHUNK), jnp.int32),
         pltpu.VMEM((CHUNK, D), jnp.float32),
         pltpu.VMEM((CHUNK, D), jnp.float32)] * 2
        + [pltpu.VMEM_SHARED((N, D), jnp.float32)]
        + [pltpu.SemaphoreType.DMA] * 8
    ),
)(_sc_body)


def _post_body(nso_ref, a0_ref, a1_ref, w_ref, o_ref):
    agg = a0_ref[...] + a1_ref[...]
    o_ref[...] = (COS * nso_ref[...]
                  + jnp.dot(agg, w_ref[...],
                            preferred_element_type=jnp.float32))


def _post(nso, agg0, agg1, W_out_scaled):
    B = 1000
    return pl.pallas_call(
        _post_body,
        grid=(N // B,),
        in_specs=[
            pl.BlockSpec((B, DOUT), lambda i: (i, 0)),
            pl.BlockSpec((B, D), lambda i: (i, 0)),
            pl.BlockSpec((B, D), lambda i: (i, 0)),
            pl.BlockSpec((D, DOUT), lambda i: (0, 0)),
        ],
        out_specs=pl.BlockSpec((B, DOUT), lambda i: (i, 0)),
        out_shape=jax.ShapeDtypeStruct((N, DOUT), jnp.float32),
    )(nso, agg0, agg1, W_out_scaled)


def kernel(node_input, edge_src, edge_dst, edge_attr, edge_scalar_attr,
           W_self, W_mlp1, W_mlp2, W_tp, W_out):
    nf, nso = _self_interaction(node_input, W_self)

    eye = jnp.eye(H, dtype=jnp.bfloat16)
    rep = jnp.repeat(eye, DE, axis=1)
    til = jnp.tile(jnp.eye(DE, dtype=jnp.bfloat16), (1, H))
    W2s = (jnp.transpose(W_tp, (0, 2, 1)).reshape(H * DE, D)
           * INV_SQRT_H).astype(jnp.bfloat16)
    W1b = jnp.kron(jnp.eye(8, dtype=jnp.float32), W_mlp1)
    W2b = jnp.kron(jnp.eye(8, dtype=jnp.float32), W_mlp2)

    mix = _edge_mix(edge_scalar_attr.reshape(E // 8, 128),
                    edge_attr.reshape(E // 8, 128),
                    W1b, W2b, rep, til, W2s)

    MB = 6400
    perm = jnp.arange(E, dtype=jnp.int32).reshape(
        E // MB, MB // 8, 8).transpose(0, 2, 1).reshape(E)
    src4d = edge_src[perm].reshape(NW, NCHUNK, 1, CHUNK)
    dst4d = edge_dst[perm].reshape(NW, NCHUNK, 1, CHUNK)
    aggs = _sc_scatter(nf, mix, src4d, dst4d)

    return _post(nso, aggs[0], aggs[1], W_out * (SIN * INV_SQRT_NN))


# speedup vs baseline: 1.4875x; 1.1187x over previous
import functools
import math

import jax
import jax.numpy as jnp
from jax import lax
from jax.experimental import pallas as pl
from jax.experimental.pallas import tpu as pltpu
from jax.experimental.pallas import tpu_sc as plsc

N = 10000
E = 320000
D = 128
DE = 16
H = 16
DOUT = 128
COS = math.cos(math.pi / 8)
SIN = math.sin(math.pi / 8)
INV_SQRT_NN = 1.0 / math.sqrt(32.0)
INV_SQRT_H = 1.0 / math.sqrt(float(H))

NC, NS = 2, 16
NW = NC * NS
EPW = E // NW
CHUNK = 40
NCHUNK = EPW // CHUNK
ZROWS = 40
NZB = N // ZROWS


def _self_body(x_ref, w_ref, nf_ref, nso_ref):
    t = jnp.dot(x_ref[...], w_ref[...], preferred_element_type=jnp.float32)
    nf_ref[...] = t[:, :D]
    nso_ref[...] = t[:, D:]


def _self_interaction(node_input, W_self):
    B = 1000
    return pl.pallas_call(
        _self_body,
        grid=(N // B,),
        in_specs=[
            pl.BlockSpec((B, D), lambda i: (i, 0)),
            pl.BlockSpec((D, D + DOUT), lambda i: (0, 0)),
        ],
        out_specs=[
            pl.BlockSpec((B, D), lambda i: (i, 0)),
            pl.BlockSpec((B, DOUT), lambda i: (i, 0)),
        ],
        out_shape=[
            jax.ShapeDtypeStruct((N, D), jnp.float32),
            jax.ShapeDtypeStruct((N, DOUT), jnp.float32),
        ],
    )(node_input, W_self)


def _mix_body(esa_ref, ea_ref, w1_ref, w2_ref, rep_ref, til_ref, wtp_ref,
              mix_ref):
    rows = esa_ref.shape[0]
    w = jax.nn.gelu(jnp.dot(esa_ref[...], w1_ref[...],
                            preferred_element_type=jnp.float32))
    w = jax.nn.gelu(jnp.dot(w, w2_ref[...],
                            preferred_element_type=jnp.float32))
    ea = ea_ref[...]
    for i in range(8):
        w_i = w[:, 16 * i:16 * i + 16].astype(jnp.bfloat16)
        ea_i = ea[:, 16 * i:16 * i + 16].astype(jnp.bfloat16)
        a_i = (jnp.dot(w_i, rep_ref[...], preferred_element_type=jnp.float32)
               * jnp.dot(ea_i, til_ref[...],
                         preferred_element_type=jnp.float32)
               ).astype(jnp.bfloat16)
        mix_ref[rows * i:rows * (i + 1), :] = jnp.dot(
            a_i, wtp_ref[...], preferred_element_type=jnp.float32)


def _edge_mix(esa_p, ea_p, W1b, W2b, rep, til, W2s):
    B = 6400
    return pl.pallas_call(
        _mix_body,
        grid=(E // B,),
        in_specs=[
            pl.BlockSpec((B // 8, 128), lambda i: (i, 0)),
            pl.BlockSpec((B // 8, 128), lambda i: (i, 0)),
            pl.BlockSpec((128, 128), lambda i: (0, 0)),
            pl.BlockSpec((128, 128), lambda i: (0, 0)),
            pl.BlockSpec((H, H * DE), lambda i: (0, 0)),
            pl.BlockSpec((DE, H * DE), lambda i: (0, 0)),
            pl.BlockSpec((H * DE, D), lambda i: (0, 0)),
        ],
        out_specs=pl.BlockSpec((B, D), lambda i: (i, 0)),
        out_shape=jax.ShapeDtypeStruct((E, D), jnp.float32),
    )(esa_p, ea_p, W1b, W2b, rep, til, W2s)


def _sc_body(nf_hbm, mix_hbm, src_hbm, dst_hbm, out_hbm,
             src_v0, src_v1, src_v2, src_v3, dst_v0, dst_v1, dst_v2, dst_v3,
             rows_v0, rows_v1, rows_v2, rows_v3, mix_v0, mix_v1,
             agg_sh, sem_s0, sem_s1, sem_s2, sem_s3,
             sem_d0, sem_d1, sem_d2, sem_d3,
             sem_g0, sem_g1, sem_g2, sem_g3, sem_m0, sem_m1):
    c = lax.axis_index("c")
    s = lax.axis_index("s")
    wid = c * NS + s

    src_b = [src_v0, src_v1, src_v2, src_v3]
    dst_b = [dst_v0, dst_v1, dst_v2, dst_v3]
    rows_b = [rows_v0, rows_v1, rows_v2, rows_v3]
    mix_b = [mix_v0, mix_v1]
    ss = [sem_s0, sem_s1, sem_s2, sem_s3]
    sd = [sem_d0, sem_d1, sem_d2, sem_d3]
    sg = [sem_g0, sem_g1, sem_g2, sem_g3]
    sm = [sem_m0, sem_m1]

    def zrow(r, carry):
        for g in range(D // 16):
            rows_v0[r, pl.ds(16 * g, 16)] = jnp.zeros((16,), jnp.float32)
        return carry

    lax.fori_loop(0, ZROWS, zrow, 0)
    for k in range((NZB + NS - 1) // NS):
        b = s + NS * k
        @pl.when(b < NZB)
        def _():
            pltpu.sync_copy(rows_v0, agg_sh.at[pl.ds(b * ZROWS, ZROWS)])
    plsc.subcore_barrier()

    def issue_idx(j, p):
        pltpu.async_copy(src_hbm.at[wid, j], src_b[p], ss[p])
        pltpu.async_copy(dst_hbm.at[wid, j], dst_b[p], sd[p])

    def issue_mix(j, m):
        pltpu.async_copy(mix_hbm.at[pl.ds(wid * EPW + j * CHUNK, CHUNK)],
                         mix_b[m], sm[m])

    def issue_gather(p):
        pltpu.make_async_copy(src_hbm.at[wid, 0], src_b[p], ss[p]).wait()
        pltpu.async_copy(nf_hbm.at[src_b[p].at[0]], rows_b[p], sg[p])

    for k in range(4):
        issue_idx(k, k)
    issue_mix(0, 0)
    issue_mix(1, 1)
    issue_gather(0)
    issue_gather(1)

    def sub(j, p, m):
        @pl.when(j + 2 < NCHUNK)
        def _():
            issue_gather(p ^ 2)
        pltpu.make_async_copy(nf_hbm.at[src_b[p].at[0]], rows_b[p],
                              sg[p]).wait()
        pltpu.make_async_copy(mix_hbm.at[pl.ds(0, CHUNK)], mix_b[m],
                              sm[m]).wait()

        def mrow(r, inner):
            for g in range(D // 16):
                sl = pl.ds(16 * g, 16)
                rows_b[p][r, sl] = rows_b[p][r, sl] * mix_b[m][r, sl]
            return inner

        lax.fori_loop(0, CHUNK, mrow, 0)
        pltpu.make_async_copy(dst_hbm.at[wid, 0], dst_b[p], sd[p]).wait()
        pltpu.sync_copy(rows_b[p], agg_sh.at[dst_b[p].at[0]], add=True)
        @pl.when(j + 4 < NCHUNK)
        def _():
            issue_idx(j + 4, p)
        @pl.when(j + 2 < NCHUNK)
        def _():
            issue_mix(j + 2, m)

    def quad(t, carry):
        j = 4 * t
        for u in range(4):
            @pl.when(j + u < NCHUNK)
            def _():
                sub(j + u, u, u % 2)
        return carry

    lax.fori_loop(0, (NCHUNK + 3) // 4, quad, 0)
    plsc.subcore_barrier()

    for k in range((NZB + NS - 1) // NS):
        b = s + NS * k
        @pl.when(b < NZB)
        def _():
            pltpu.sync_copy(agg_sh.at[pl.ds(b * ZROWS, ZROWS)], rows_v0)
            pltpu.sync_copy(rows_v0, out_hbm.at[c, pl.ds(b * ZROWS, ZROWS)])


_sc_scatter = functools.partial(
    pl.kernel,
    out_type=jax.ShapeDtypeStruct((NC, N, D), jnp.float32),
    mesh=plsc.VectorSubcoreMesh(core_axis_name="c", subcore_axis_name="s"),
    scratch_types=(
        [pltpu.VMEM((1, CHUNK), jnp.int32)] * 8
        + [pltpu.VMEM((CHUNK, D), jnp.float32)] * 6
        + [pltpu.VMEM_SHARED((N, D), jnp.float32)]
        + [pltpu.SemaphoreType.DMA] * 14
    ),
)(_sc_body)


def _post_body(nso_ref, a0_ref, a1_ref, w_ref, o_ref):
    agg = a0_ref[...] + a1_ref[...]
    o_ref[...] = (COS * nso_ref[...]
                  + jnp.dot(agg, w_ref[...],
                            preferred_element_type=jnp.float32))


def _post(nso, agg0, agg1, W_out_scaled):
    B = 1000
    return pl.pallas_call(
        _post_body,
        grid=(N // B,),
        in_specs=[
            pl.BlockSpec((B, DOUT), lambda i: (i, 0)),
            pl.BlockSpec((B, D), lambda i: (i, 0)),
            pl.BlockSpec((B, D), lambda i: (i, 0)),
            pl.BlockSpec((D, DOUT), lambda i: (0, 0)),
        ],
        out_specs=pl.BlockSpec((B, DOUT), lambda i: (i, 0)),
        out_shape=jax.ShapeDtypeStruct((N, DOUT), jnp.float32),
    )(nso, agg0, agg1, W_out_scaled)


def kernel(node_input, edge_src, edge_dst, edge_attr, edge_scalar_attr,
           W_self, W_mlp1, W_mlp2, W_tp, W_out):
    nf, nso = _self_interaction(node_input, W_self)

    eye = jnp.eye(H, dtype=jnp.bfloat16)
    rep = jnp.repeat(eye, DE, axis=1)
    til = jnp.tile(jnp.eye(DE, dtype=jnp.bfloat16), (1, H))
    W2s = (jnp.transpose(W_tp, (0, 2, 1)).reshape(H * DE, D)
           * INV_SQRT_H).astype(jnp.bfloat16)
    W1b = jnp.kron(jnp.eye(8, dtype=jnp.float32), W_mlp1)
    W2b = jnp.kron(jnp.eye(8, dtype=jnp.float32), W_mlp2)

    mix = _edge_mix(edge_scalar_attr.reshape(E // 8, 128),
                    edge_attr.reshape(E // 8, 128),
                    W1b, W2b, rep, til, W2s)

    MB = 6400
    perm = jnp.arange(E, dtype=jnp.int32).reshape(
        E // MB, MB // 8, 8).transpose(0, 2, 1).reshape(E)
    src4d = edge_src[perm].reshape(NW, NCHUNK, 1, CHUNK)
    dst4d = edge_dst[perm].reshape(NW, NCHUNK, 1, CHUNK)
    aggs = _sc_scatter(nf, mix, src4d, dst4d)

    return _post(nso, aggs[0], aggs[1], W_out * (SIN * INV_SQRT_NN))


# speedup vs baseline: 1.5577x; 1.0472x over previous
import functools
import math

import jax
import jax.numpy as jnp
from jax import lax
from jax.experimental import pallas as pl
from jax.experimental.pallas import tpu as pltpu
from jax.experimental.pallas import tpu_sc as plsc

N = 10000
E = 320000
D = 128
DE = 16
H = 16
DOUT = 128
COS = math.cos(math.pi / 8)
SIN = math.sin(math.pi / 8)
INV_SQRT_NN = 1.0 / math.sqrt(32.0)
INV_SQRT_H = 1.0 / math.sqrt(float(H))

NC, NS = 2, 16
NW = NC * NS
EPW = E // NW
CHUNK = 40
NCHUNK = EPW // CHUNK
ZROWS = 40
NZB = N // ZROWS


def _self_body(x_ref, w_ref, nf_ref, nso_ref):
    t = jnp.dot(x_ref[...], w_ref[...], preferred_element_type=jnp.float32)
    nf_ref[...] = t[:, :D]
    nso_ref[...] = t[:, D:]


def _self_interaction(node_input, W_self):
    B = 1000
    return pl.pallas_call(
        _self_body,
        grid=(N // B,),
        in_specs=[
            pl.BlockSpec((B, D), lambda i: (i, 0)),
            pl.BlockSpec((D, D + DOUT), lambda i: (0, 0)),
        ],
        out_specs=[
            pl.BlockSpec((B, D), lambda i: (i, 0)),
            pl.BlockSpec((B, DOUT), lambda i: (i, 0)),
        ],
        out_shape=[
            jax.ShapeDtypeStruct((N, D), jnp.float32),
            jax.ShapeDtypeStruct((N, DOUT), jnp.float32),
        ],
    )(node_input, W_self)


def _mix_body(esa_ref, ea_ref, w1_ref, w2_ref, rep_ref, til_ref, wtp_ref,
              mix_ref):
    w = jax.nn.gelu(jnp.dot(esa_ref[...], w1_ref[...],
                            preferred_element_type=jnp.float32))
    w = jax.nn.gelu(jnp.dot(w, w2_ref[...],
                            preferred_element_type=jnp.float32))
    a = (jnp.dot(w.astype(jnp.bfloat16), rep_ref[...],
                 preferred_element_type=jnp.float32)
         * jnp.dot(ea_ref[...].astype(jnp.bfloat16), til_ref[...],
                   preferred_element_type=jnp.float32)).astype(jnp.bfloat16)
    mix_ref[...] = jnp.dot(a, wtp_ref[...],
                           preferred_element_type=jnp.float32)


def _edge_mix(esa, ea, W_mlp1, W_mlp2, rep, til, W2s):
    B = 6400
    return pl.pallas_call(
        _mix_body,
        grid=(E // B,),
        in_specs=[
            pl.BlockSpec((B, DE), lambda i: (i, 0)),
            pl.BlockSpec((B, DE), lambda i: (i, 0)),
            pl.BlockSpec((DE, H), lambda i: (0, 0)),
            pl.BlockSpec((H, H), lambda i: (0, 0)),
            pl.BlockSpec((H, H * DE), lambda i: (0, 0)),
            pl.BlockSpec((DE, H * DE), lambda i: (0, 0)),
            pl.BlockSpec((H * DE, D), lambda i: (0, 0)),
        ],
        out_specs=pl.BlockSpec((B, D), lambda i: (i, 0)),
        out_shape=jax.ShapeDtypeStruct((E, D), jnp.float32),
    )(esa, ea, W_mlp1, W_mlp2, rep, til, W2s)


def _sc_body(nf_hbm, mix_hbm, src_hbm, dst_hbm, out_hbm,
             src_v0, src_v1, src_v2, src_v3, dst_v0, dst_v1, dst_v2, dst_v3,
             rows_v0, rows_v1, rows_v2, rows_v3, mix_v0, mix_v1,
             agg_sh, sem_s0, sem_s1, sem_s2, sem_s3,
             sem_d0, sem_d1, sem_d2, sem_d3,
             sem_g0, sem_g1, sem_g2, sem_g3, sem_m0, sem_m1):
    c = lax.axis_index("c")
    s = lax.axis_index("s")
    wid = c * NS + s

    src_b = [src_v0, src_v1, src_v2, src_v3]
    dst_b = [dst_v0, dst_v1, dst_v2, dst_v3]
    rows_b = [rows_v0, rows_v1, rows_v2, rows_v3]
    mix_b = [mix_v0, mix_v1]
    ss = [sem_s0, sem_s1, sem_s2, sem_s3]
    sd = [sem_d0, sem_d1, sem_d2, sem_d3]
    sg = [sem_g0, sem_g1, sem_g2, sem_g3]
    sm = [sem_m0, sem_m1]

    def zrow(r, carry):
        for g in range(D // 16):
            rows_v0[r, pl.ds(16 * g, 16)] = jnp.zeros((16,), jnp.float32)
        return carry

    lax.fori_loop(0, ZROWS, zrow, 0)
    for k in range((NZB + NS - 1) // NS):
        b = s + NS * k
        @pl.when(b < NZB)
        def _():
            pltpu.sync_copy(rows_v0, agg_sh.at[pl.ds(b * ZROWS, ZROWS)])
    plsc.subcore_barrier()

    def issue_idx(j, p):
        pltpu.async_copy(src_hbm.at[wid, j], src_b[p], ss[p])
        pltpu.async_copy(dst_hbm.at[wid, j], dst_b[p], sd[p])

    def issue_mix(j, m):
        pltpu.async_copy(mix_hbm.at[pl.ds(wid * EPW + j * CHUNK, CHUNK)],
                         mix_b[m], sm[m])

    def issue_gather(p):
        pltpu.make_async_copy(src_hbm.at[wid, 0], src_b[p], ss[p]).wait()
        pltpu.async_copy(nf_hbm.at[src_b[p].at[0]], rows_b[p], sg[p])

    for k in range(4):
        issue_idx(k, k)
    issue_mix(0, 0)
    issue_mix(1, 1)
    issue_gather(0)
    issue_gather(1)

    def sub(j, p, m):
        @pl.when(j + 2 < NCHUNK)
        def _():
            issue_gather(p ^ 2)
        pltpu.make_async_copy(nf_hbm.at[src_b[p].at[0]], rows_b[p],
                              sg[p]).wait()
        pltpu.make_async_copy(mix_hbm.at[pl.ds(0, CHUNK)], mix_b[m],
                              sm[m]).wait()

        def mrow(r, inner):
            for g in range(D // 16):
                sl = pl.ds(16 * g, 16)
                rows_b[p][r, sl] = rows_b[p][r, sl] * mix_b[m][r, sl]
            return inner

        lax.fori_loop(0, CHUNK, mrow, 0)
        pltpu.make_async_copy(dst_hbm.at[wid, 0], dst_b[p], sd[p]).wait()
        pltpu.sync_copy(rows_b[p], agg_sh.at[dst_b[p].at[0]], add=True)
        @pl.when(j + 4 < NCHUNK)
        def _():
            issue_idx(j + 4, p)
        @pl.when(j + 2 < NCHUNK)
        def _():
            issue_mix(j + 2, m)

    def quad(t, carry):
        j = 4 * t
        for u in range(4):
            @pl.when(j + u < NCHUNK)
            def _():
                sub(j + u, u, u % 2)
        return carry

    lax.fori_loop(0, (NCHUNK + 3) // 4, quad, 0)
    plsc.subcore_barrier()

    for k in range((NZB + NS - 1) // NS):
        b = s + NS * k
        @pl.when(b < NZB)
        def _():
            pltpu.sync_copy(agg_sh.at[pl.ds(b * ZROWS, ZROWS)], rows_v0)
            pltpu.sync_copy(rows_v0, out_hbm.at[c, pl.ds(b * ZROWS, ZROWS)])


_sc_scatter = functools.partial(
    pl.kernel,
    out_type=jax.ShapeDtypeStruct((NC, N, D), jnp.float32),
    mesh=plsc.VectorSubcoreMesh(core_axis_name="c", subcore_axis_name="s"),
    scratch_types=(
        [pltpu.VMEM((1, CHUNK), jnp.int32)] * 8
        + [pltpu.VMEM((CHUNK, D), jnp.float32)] * 6
        + [pltpu.VMEM_SHARED((N, D), jnp.float32)]
        + [pltpu.SemaphoreType.DMA] * 14
    ),
)(_sc_body)


def _post_body(nso_ref, a0_ref, a1_ref, w_ref, o_ref):
    agg = a0_ref[...] + a1_ref[...]
    o_ref[...] = (COS * nso_ref[...]
                  + jnp.dot(agg, w_ref[...],
                            preferred_element_type=jnp.float32))


def _post(nso, agg0, agg1, W_out_scaled):
    B = 1000
    return pl.pallas_call(
        _post_body,
        grid=(N // B,),
        in_specs=[
            pl.BlockSpec((B, DOUT), lambda i: (i, 0)),
            pl.BlockSpec((B, D), lambda i: (i, 0)),
            pl.BlockSpec((B, D), lambda i: (i, 0)),
            pl.BlockSpec((D, DOUT), lambda i: (0, 0)),
        ],
        out_specs=pl.BlockSpec((B, DOUT), lambda i: (i, 0)),
        out_shape=jax.ShapeDtypeStruct((N, DOUT), jnp.float32),
    )(nso, agg0, agg1, W_out_scaled)


def kernel(node_input, edge_src, edge_dst, edge_attr, edge_scalar_attr,
           W_self, W_mlp1, W_mlp2, W_tp, W_out):
    nf, nso = _self_interaction(node_input, W_self)

    eye = jnp.eye(H, dtype=jnp.bfloat16)
    rep = jnp.repeat(eye, DE, axis=1)
    til = jnp.tile(jnp.eye(DE, dtype=jnp.bfloat16), (1, H))
    W2s = (jnp.transpose(W_tp, (0, 2, 1)).reshape(H * DE, D)
           * INV_SQRT_H).astype(jnp.bfloat16)

    mix = _edge_mix(edge_scalar_attr, edge_attr, W_mlp1, W_mlp2, rep, til,
                    W2s)

    src4d = edge_src.reshape(NW, NCHUNK, 1, CHUNK)
    dst4d = edge_dst.reshape(NW, NCHUNK, 1, CHUNK)
    aggs = _sc_scatter(nf, mix, src4d, dst4d)

    return _post(nso, aggs[0], aggs[1], W_out * (SIN * INV_SQRT_NN))
